# Initial kernel scaffold; baseline (speedup 1.0000x reference)
import jax
import jax.numpy as jnp
from jax.experimental import pallas as pl


def kernel(x, edge_index, W, att_src, att_dst, bias):
    raise NotImplementedError("write your pallas kernel here")


# speedup vs baseline: 6.1027x; 6.1027x over previous
import functools

import jax
import jax.numpy as jnp
from jax import lax
from jax.experimental import pallas as pl
from jax.experimental.pallas import tpu as pltpu
from jax.experimental.pallas import tpu_sc as plsc

NEG_SLOPE = 0.2
LANE = 16
TRASH = 128
PACK = 14


def _prep_body(x_ref, w_ref, asw_ref, adw_ref, xs_ref, asrc_ref, adst_ref, g_ref):
    xs = jnp.dot(x_ref[...], w_ref[...], preferred_element_type=jnp.float32)
    xs_ref[...] = xs
    a_s = jnp.sum(xs * asw_ref[...], axis=1, keepdims=True)
    a_d = jnp.sum(xs * adw_ref[...], axis=1, keepdims=True)
    asrc_ref[...] = a_s
    adst_ref[...] = a_d
    gg = jnp.max(a_s) + jnp.max(a_d)
    g_ref[...] = jnp.full((1, 1), jnp.where(gg >= 0.0, gg, NEG_SLOPE * gg),
                          dtype=jnp.float32)


def _tc_prep(x, w, att_src_row, att_dst_row):
    n = x.shape[0]
    c = w.shape[1]
    return pl.pallas_call(
        _prep_body,
        out_shape=[
            jax.ShapeDtypeStruct((n, c), jnp.float32),
            jax.ShapeDtypeStruct((n, 1), jnp.float32),
            jax.ShapeDtypeStruct((n, 1), jnp.float32),
            jax.ShapeDtypeStruct((1, 1), jnp.float32),
        ],
    )(x, w, att_src_row, att_dst_row)


def _comb_body(p0_ref, p1_ref, b_ref, o_ref):
    o_ref[...] = (jnp.concatenate([p0_ref[...], p1_ref[...]], axis=0)
                  + b_ref[...])


def _tc_combine(p0, p1, brow):
    return pl.pallas_call(
        _comb_body,
        out_shape=jax.ShapeDtypeStruct((2 * p0.shape[0], p0.shape[1]),
                                       jnp.float32),
    )(p0, p1, brow)


def _make_sc_kernel(half, nj, d):
    n_logit = 2 * half + TRASH
    glrow = n_logit // 128
    cap = 2 * nj * 128
    drows = n_logit // 16
    arows = (half + TRASH) // 16
    orows = half // 16
    parts = [(0, nj)] if nj <= 48 else [(0, 48), (48, nj - 48)]
    njb = parts[0][1]
    mesh = plsc.VectorSubcoreMesh(core_axis_name="c", subcore_axis_name="s",
                                  num_cores=2, num_subcores=16)

    @functools.partial(
        pl.kernel,
        out_type=[
            jax.ShapeDtypeStruct((32 * nj, 128), jnp.float32),
            jax.ShapeDtypeStruct((half, d), jnp.float32),
            jax.ShapeDtypeStruct((half, d), jnp.float32),
        ],
        mesh=mesh,
        scratch_types=[
            pltpu.VMEM((glrow + 1, 128), jnp.float32),
            pltpu.VMEM((glrow, 128), jnp.float32),
            pltpu.VMEM((njb, 128), jnp.int32),
            pltpu.VMEM((njb, 128), jnp.int32),
            pltpu.VMEM((1, 128), jnp.float32),
            pltpu.VMEM((8, 128), jnp.float32),
            pltpu.VMEM((cap,), jnp.int32),
            pltpu.VMEM((n_logit,), jnp.float32),
            pltpu.VMEM((128, d), jnp.float32),
            pltpu.VMEM((128,), jnp.float32),
            pltpu.VMEM((1, 128), jnp.int32),
            pltpu.VMEM((128,), jnp.int32),
            pltpu.VMEM_SHARED((n_logit,), jnp.float32),
            pltpu.VMEM_SHARED((half + TRASH, d), jnp.float32),
            pltpu.SemaphoreType.DMA,
        ],
        compiler_params=pltpu.CompilerParams(needs_layout_passes=False),
    )
    def sc_gat(src_hbm, dst_hbm, asrc_hbm, adst_hbm, xs_hbm,
               att_hbm, outp0_hbm, outp1_hbm, asrc_v, adst_v, src_v, dst_v,
               prow_v, att8_v, packedC, denom_v, rows_v, attw_v, dstRow_v,
               srcRow_v, denom_sh, out_sh, sem):
        c = lax.axis_index("c")
        s = lax.axis_index("s")
        zv = jnp.zeros((LANE,), jnp.float32)

        def body_zd(i, carry):
            denom_v[pl.ds(i * LANE, LANE)] = zv
            return carry

        lax.fori_loop(0, n_logit // LANE, body_zd, 0)

        def body_zr(r, carry):
            for i in range(d // LANE):
                rows_v[r, pl.ds(i * LANE, LANE)] = zv
            return carry

        lax.fori_loop(0, 128, body_zr, 0)
        pltpu.sync_copy(denom_v.at[pl.ds(s * drows, drows)],
                        denom_sh.at[pl.ds(s * drows, drows)])
        off = 0
        while off < arows:
            m = min(128, arows - off)
            pltpu.sync_copy(rows_v.at[pl.ds(0, m)],
                            out_sh.at[pl.ds(s * arows + off, m)])
            off += m
        pltpu.sync_copy(asrc_hbm, asrc_v)
        pltpu.sync_copy(adst_hbm, adst_v)
        plsc.subcore_barrier()
        g_vec = asrc_v[glrow, pl.ds(0, LANE)]

        def edge_p(sv, dv):
            a = plsc.load_gather(asrc_v, [sv >> 7, sv & 127])
            b = plsc.load_gather(adst_v, [dv >> 7, dv & 127])
            al = a + b
            al = jnp.where(al >= 0.0, al, al * NEG_SLOPE)
            return jnp.exp(al - g_vec)

        cnt = jnp.int32(0)
        for which in range(2):
            gchunk = 2 * s + (1 - c) if which == 0 else 2 * s + c
            for off_r, rr in parts:
                pltpu.sync_copy(src_hbm.at[pl.ds(gchunk * nj + off_r, rr)],
                                src_v.at[pl.ds(0, rr)])
                pltpu.sync_copy(dst_hbm.at[pl.ds(gchunk * nj + off_r, rr)],
                                dst_v.at[pl.ds(0, rr)])

                def body_j(j, cnt):
                    for i in range(128 // LANE):
                        sl = pl.ds(i * LANE, LANE)
                        sv = src_v[j, sl]
                        dv = dst_v[j, sl]
                        prow_v[0, sl] = edge_p(sv, dv)
                        dvr = dv - c * half
                        keep = (dvr >= 0) & (dvr < half)
                        plsc.store_compressed(packedC.at[pl.ds(cnt, LANE)],
                                              sv | (dvr << PACK), mask=keep)
                        cnt = cnt + plsc.all_reduce_population_count(keep)[0]
                    pltpu.sync_copy(prow_v.at[0], denom_sh.at[dst_v.at[j]],
                                    add=True)
                    return cnt

                cnt = lax.fori_loop(0, rr, body_j, cnt)

        iota16 = lax.iota(jnp.int32, LANE)
        rem = (128 - (cnt & 127)) & 127

        def body_pad(i, carry):
            cnt, rem = carry
            t = jnp.minimum(rem, LANE)
            plsc.store_compressed(packedC.at[pl.ds(cnt, LANE)],
                                  (half + iota16) << PACK,
                                  mask=iota16 < t)
            return (cnt + t, rem - t)

        cnt, _ = lax.fori_loop(0, 8, body_pad, (cnt, rem))
        ngrp = cnt >> 7
        plsc.subcore_barrier()

        pltpu.sync_copy(denom_sh, denom_v)

        def body_rcp(i, carry):
            sl = pl.ds(i * LANE, LANE)
            denom_v[sl] = 1.0 / denom_v[sl]
            return carry

        lax.fori_loop(0, n_logit // LANE, body_rcp, 0)

        for off_r, rr in parts:
            pltpu.sync_copy(
                src_hbm.at[pl.ds((2 * s + c) * nj + off_r, rr)],
                src_v.at[pl.ds(0, rr)])
            pltpu.sync_copy(
                dst_hbm.at[pl.ds((2 * s + c) * nj + off_r, rr)],
                dst_v.at[pl.ds(0, rr)])

            def body_jo(jo, carry):
                for jj in range(8):
                    j = jo * 8 + jj
                    for i in range(128 // LANE):
                        sl = pl.ds(i * LANE, LANE)
                        sv = src_v[j, sl]
                        dv = dst_v[j, sl]
                        r = plsc.load_gather(denom_v, [dv])
                        att8_v[jj, sl] = edge_p(sv, dv) * r
                pltpu.sync_copy(
                    att8_v,
                    att_hbm.at[pl.ds((2 * s + c) * nj + off_r + jo * 8, 8)])
                return carry

            lax.fori_loop(0, rr // 8, body_jo, 0)

        def body_grp(g2, carry):
            base = g2 * 128
            for i in range(128 // LANE):
                sl = pl.ds(i * LANE, LANE)
                pk = packedC[pl.ds(base + i * LANE, LANE)]
                sC = pk & ((1 << PACK) - 1)
                dvr = pk >> PACK
                srcRow_v[sl] = sC
                dstRow_v[0, sl] = dvr
                dabs = dvr + c * half
                r = plsc.load_gather(denom_v, [dabs])
                attw_v[sl] = edge_p(sC, dabs) * r
            pltpu.async_copy(xs_hbm.at[srcRow_v], rows_v, sem).wait()

            def body_e(e, ecarry):
                ab = plsc.load_gather(attw_v,
                                      [jnp.full((LANE,), e, jnp.int32)])
                for f in range(d // LANE):
                    slf = pl.ds(f * LANE, LANE)
                    rows_v[e, slf] = rows_v[e, slf] * ab
                return ecarry

            lax.fori_loop(0, 128, body_e, 0)
            pltpu.sync_copy(rows_v, out_sh.at[dstRow_v.at[0]], add=True)
            return carry

        lax.fori_loop(0, ngrp, body_grp, 0)
        plsc.subcore_barrier()

        off = 0
        while off < orows:
            m = min(128, orows - off)
            pltpu.sync_copy(out_sh.at[pl.ds(s * orows + off, m)],
                            rows_v.at[pl.ds(0, m)])

            @pl.when(c == 0)
            def _(off=off, m=m):
                pltpu.sync_copy(rows_v.at[pl.ds(0, m)],
                                outp0_hbm.at[pl.ds(s * orows + off, m)])

            @pl.when(c == 1)
            def _(off=off, m=m):
                pltpu.sync_copy(rows_v.at[pl.ds(0, m)],
                                outp1_hbm.at[pl.ds(s * orows + off, m)])

            off += m

    return sc_gat


def kernel(x, edge_index, W, att_src, att_dst, bias):
    n, din = x.shape
    c = W.shape[1]
    e = edge_index.shape[1]
    ep = e + n
    nj = -(-ep // (32 * 128))
    nj = -(-nj // 8) * 8
    e_pad = 32 * nj * 128
    half = -(-(n // 2 + 1) // 128) * 128
    if 2 * half <= n:
        half += 128
    n_acc = 2 * half

    loops = jnp.arange(n, dtype=edge_index.dtype)
    ei = jnp.concatenate([edge_index, jnp.stack([loops, loops])], axis=1)
    pad = e_pad - ep
    src = jnp.concatenate([ei[0], jnp.zeros((pad,), jnp.int32)])
    pad_dst = n + (jnp.arange(pad, dtype=jnp.int32) % (n_acc - n))
    dst = jnp.concatenate([ei[1], pad_dst])
    src2 = src.reshape(e_pad // 128, 128)
    dst2 = dst.reshape(e_pad // 128, 128)

    xs, asrc, adst, g = _tc_prep(x, W, att_src.reshape(1, c),
                                 att_dst.reshape(1, c))
    n_logit = n_acc + TRASH
    asrc_p = jnp.concatenate(
        [jnp.pad(asrc[:, 0], (0, n_logit - n)),
         jnp.broadcast_to(g.reshape(1), (128,))]).reshape(
             n_logit // 128 + 1, 128)
    adst_p = jnp.pad(adst[:, 0], (0, n_logit - n)).reshape(n_logit // 128, 128)

    sc_gat = _make_sc_kernel(half, nj, c)
    att2, outp0, outp1 = sc_gat(src2, dst2, asrc_p, adst_p, xs)

    out = _tc_combine(outp0, outp1, bias.reshape(1, c))[:n]
    att = att2.reshape(e_pad)[:ep].reshape(ep, 1)
    return (out, (ei, att))


# speedup vs baseline: 22.8770x; 3.7487x over previous
import functools

import jax
import jax.numpy as jnp
from jax import lax
from jax.experimental import pallas as pl
from jax.experimental.pallas import tpu as pltpu
from jax.experimental.pallas import tpu_sc as plsc

NEG_SLOPE = 0.2
LANE = 16
TRASH = 128
PACK = 14


def _prep_body(x_ref, w_ref, asw_ref, adw_ref, xs_ref, asrc_ref, adst_ref, g_ref):
    xs = jnp.dot(x_ref[...], w_ref[...], preferred_element_type=jnp.float32)
    xs_ref[...] = xs
    a_s = jnp.sum(xs * asw_ref[...], axis=1, keepdims=True)
    a_d = jnp.sum(xs * adw_ref[...], axis=1, keepdims=True)
    asrc_ref[...] = a_s
    adst_ref[...] = a_d
    gg = jnp.max(a_s) + jnp.max(a_d)
    g_ref[...] = jnp.full((1, 1), jnp.where(gg >= 0.0, gg, NEG_SLOPE * gg),
                          dtype=jnp.float32)


def _tc_prep(x, w, att_src_row, att_dst_row):
    n = x.shape[0]
    c = w.shape[1]
    return pl.pallas_call(
        _prep_body,
        out_shape=[
            jax.ShapeDtypeStruct((n, c), jnp.float32),
            jax.ShapeDtypeStruct((n, 1), jnp.float32),
            jax.ShapeDtypeStruct((n, 1), jnp.float32),
            jax.ShapeDtypeStruct((1, 1), jnp.float32),
        ],
    )(x, w, att_src_row, att_dst_row)


def _comb_body(p0_ref, p1_ref, b_ref, o_ref):
    o_ref[...] = (jnp.concatenate([p0_ref[...], p1_ref[...]], axis=0)
                  + b_ref[...])


def _tc_combine(p0, p1, brow):
    return pl.pallas_call(
        _comb_body,
        out_shape=jax.ShapeDtypeStruct((2 * p0.shape[0], p0.shape[1]),
                                       jnp.float32),
    )(p0, p1, brow)


def _make_sc_kernel(half, nj, d):
    n_logit = 2 * half + TRASH
    glrow = n_logit // 128
    cap = 2 * nj * 128
    drows = n_logit // 16
    arows = (half + TRASH) // 16
    orows = half // 16
    parts = [(0, nj)] if nj <= 48 else [(0, 48), (48, nj - 48)]
    njb = parts[0][1]
    mesh = plsc.VectorSubcoreMesh(core_axis_name="c", subcore_axis_name="s",
                                  num_cores=2, num_subcores=16)

    @functools.partial(
        pl.kernel,
        out_type=[
            jax.ShapeDtypeStruct((32 * nj, 128), jnp.float32),
            jax.ShapeDtypeStruct((half, d), jnp.float32),
            jax.ShapeDtypeStruct((half, d), jnp.float32),
        ],
        mesh=mesh,
        scratch_types=[
            pltpu.VMEM((glrow + 1, 128), jnp.float32),
            pltpu.VMEM((glrow, 128), jnp.float32),
            pltpu.VMEM((njb, 128), jnp.int32),
            pltpu.VMEM((njb, 128), jnp.int32),
            pltpu.VMEM((1, 128), jnp.float32),
            pltpu.VMEM((8, 128), jnp.float32),
            pltpu.VMEM((cap,), jnp.int32),
            pltpu.VMEM((n_logit,), jnp.float32),
            pltpu.VMEM((128, d), jnp.float32),
            pltpu.VMEM((128,), jnp.float32),
            pltpu.VMEM((1, 128), jnp.int32),
            pltpu.VMEM((128,), jnp.int32),
            pltpu.VMEM_SHARED((n_logit,), jnp.float32),
            pltpu.VMEM_SHARED((half + TRASH, d), jnp.float32),
            pltpu.SemaphoreType.DMA,
        ],
        compiler_params=pltpu.CompilerParams(needs_layout_passes=False),
    )
    def sc_gat(src_hbm, dst_hbm, asrc_hbm, adst_hbm, xs_hbm,
               att_hbm, outp0_hbm, outp1_hbm, asrc_v, adst_v, src_v, dst_v,
               prow_v, att8_v, packedC, denom_v, rows_v, attw_v, dstRow_v,
               srcRow_v, denom_sh, out_sh, sem):
        c = lax.axis_index("c")
        s = lax.axis_index("s")
        zv = jnp.zeros((LANE,), jnp.float32)

        def body_zd(i, carry):
            denom_v[pl.ds(i * LANE, LANE)] = zv
            return carry

        lax.fori_loop(0, n_logit // LANE, body_zd, 0)

        def body_zr(r, carry):
            for i in range(d // LANE):
                rows_v[r, pl.ds(i * LANE, LANE)] = zv
            return carry

        lax.fori_loop(0, 128, body_zr, 0)
        pltpu.sync_copy(denom_v.at[pl.ds(s * drows, drows)],
                        denom_sh.at[pl.ds(s * drows, drows)])
        off = 0
        while off < arows:
            m = min(128, arows - off)
            pltpu.sync_copy(rows_v.at[pl.ds(0, m)],
                            out_sh.at[pl.ds(s * arows + off, m)])
            off += m
        pltpu.sync_copy(asrc_hbm, asrc_v)
        pltpu.sync_copy(adst_hbm, adst_v)
        plsc.subcore_barrier()
        g_vec = asrc_v[glrow, pl.ds(0, LANE)]

        def edge_p(sv, dv):
            a = plsc.load_gather(asrc_v, [sv >> 7, sv & 127])
            b = plsc.load_gather(adst_v, [dv >> 7, dv & 127])
            al = a + b
            al = jnp.where(al >= 0.0, al, al * NEG_SLOPE)
            return jnp.exp(al - g_vec)

        cnt = jnp.int32(0)
        for which in range(2):
            gchunk = 2 * s + (1 - c) if which == 0 else 2 * s + c
            for off_r, rr in parts:
                pltpu.sync_copy(src_hbm.at[pl.ds(gchunk * nj + off_r, rr)],
                                src_v.at[pl.ds(0, rr)])
                pltpu.sync_copy(dst_hbm.at[pl.ds(gchunk * nj + off_r, rr)],
                                dst_v.at[pl.ds(0, rr)])

                def body_j(j, cnt):
                    for i in range(128 // LANE):
                        sl = pl.ds(i * LANE, LANE)
                        sv = src_v[j, sl]
                        dv = dst_v[j, sl]
                        prow_v[0, sl] = edge_p(sv, dv)
                        dvr = dv - c * half
                        keep = (dvr >= 0) & (dvr < half)
                        plsc.store_compressed(packedC.at[pl.ds(cnt, LANE)],
                                              sv | (dvr << PACK), mask=keep)
                        cnt = cnt + plsc.all_reduce_population_count(keep)[0]
                    pltpu.sync_copy(prow_v.at[0], denom_sh.at[dst_v.at[j]],
                                    add=True)
                    return cnt

                cnt = lax.fori_loop(0, rr, body_j, cnt)

        iota16 = lax.iota(jnp.int32, LANE)
        rem = (128 - (cnt & 127)) & 127

        def body_pad(i, carry):
            cnt, rem = carry
            t = jnp.minimum(rem, LANE)
            plsc.store_compressed(packedC.at[pl.ds(cnt, LANE)],
                                  (half + iota16) << PACK,
                                  mask=iota16 < t)
            return (cnt + t, rem - t)

        cnt, _ = lax.fori_loop(0, 8, body_pad, (cnt, rem))
        ngrp = cnt >> 7
        plsc.subcore_barrier()

        pltpu.sync_copy(denom_sh, denom_v)

        def body_rcp(i, carry):
            sl = pl.ds(i * LANE, LANE)
            denom_v[sl] = 1.0 / denom_v[sl]
            return carry

        lax.fori_loop(0, n_logit // LANE, body_rcp, 0)

        for off_r, rr in parts:
            pltpu.sync_copy(
                src_hbm.at[pl.ds((2 * s + c) * nj + off_r, rr)],
                src_v.at[pl.ds(0, rr)])
            pltpu.sync_copy(
                dst_hbm.at[pl.ds((2 * s + c) * nj + off_r, rr)],
                dst_v.at[pl.ds(0, rr)])

            def body_jo(jo, carry):
                for jj in range(8):
                    j = jo * 8 + jj
                    for i in range(128 // LANE):
                        sl = pl.ds(i * LANE, LANE)
                        sv = src_v[j, sl]
                        dv = dst_v[j, sl]
                        r = plsc.load_gather(denom_v, [dv])
                        att8_v[jj, sl] = edge_p(sv, dv) * r
                pltpu.sync_copy(
                    att8_v,
                    att_hbm.at[pl.ds((2 * s + c) * nj + off_r + jo * 8, 8)])
                return carry

            lax.fori_loop(0, rr // 8, body_jo, 0)

        def body_grp(g2, carry):
            base = g2 * 128
            for i in range(128 // LANE):
                sl = pl.ds(i * LANE, LANE)
                pk = packedC[pl.ds(base + i * LANE, LANE)]
                sC = pk & ((1 << PACK) - 1)
                dvr = pk >> PACK
                srcRow_v[sl] = sC
                dstRow_v[0, sl] = dvr
                dabs = dvr + c * half
                r = plsc.load_gather(denom_v, [dabs])
                attw_v[sl] = edge_p(sC, dabs) * r
            pltpu.async_copy(xs_hbm.at[srcRow_v], rows_v, sem).wait()

            def body_e(e, ecarry):
                ab = plsc.load_gather(attw_v,
                                      [jnp.full((LANE,), e, jnp.int32)])
                for f in range(d // LANE):
                    slf = pl.ds(f * LANE, LANE)
                    rows_v[e, slf] = rows_v[e, slf] * ab
                return ecarry

            lax.fori_loop(0, 128, body_e, 0)
            pltpu.sync_copy(rows_v, out_sh.at[dstRow_v.at[0]], add=True)
            return carry

        lax.fori_loop(0, ngrp, body_grp, 0)
        plsc.subcore_barrier()

        off = 0
        while off < orows:
            m = min(128, orows - off)
            pltpu.sync_copy(out_sh.at[pl.ds(s * orows + off, m)],
                            rows_v.at[pl.ds(0, m)])

            @pl.when(c == 0)
            def _(off=off, m=m):
                pltpu.sync_copy(rows_v.at[pl.ds(0, m)],
                                outp0_hbm.at[pl.ds(s * orows + off, m)])

            @pl.when(c == 1)
            def _(off=off, m=m):
                pltpu.sync_copy(rows_v.at[pl.ds(0, m)],
                                outp1_hbm.at[pl.ds(s * orows + off, m)])

            off += m

    return sc_gat


def kernel(x, edge_index, W, att_src, att_dst, bias):
    n, din = x.shape
    c = W.shape[1]
    e = edge_index.shape[1]
    ep = e + n
    nj = -(-ep // (32 * 128))
    nj = -(-nj // 8) * 8
    e_pad = 32 * nj * 128
    half = -(-(n // 2 + 1) // 128) * 128
    if 2 * half <= n:
        half += 128
    n_acc = 2 * half

    loops = jnp.arange(n, dtype=edge_index.dtype)
    ei = jnp.concatenate([edge_index, jnp.stack([loops, loops])], axis=1)
    pad = e_pad - ep
    src = jnp.concatenate([ei[0], jnp.zeros((pad,), jnp.int32)])
    pad_dst = n_acc + (jnp.arange(pad, dtype=jnp.int32) % TRASH)
    dst = jnp.concatenate([ei[1], pad_dst])
    src2 = src.reshape(e_pad // 128, 128)
    dst2 = dst.reshape(e_pad // 128, 128)

    xs, asrc, adst, g = _tc_prep(x, W, att_src.reshape(1, c),
                                 att_dst.reshape(1, c))
    n_logit = n_acc + TRASH
    asrc_p = jnp.concatenate(
        [jnp.pad(asrc[:, 0], (0, n_logit - n)),
         jnp.broadcast_to(g.reshape(1), (128,))]).reshape(
             n_logit // 128 + 1, 128)
    adst_p = jnp.pad(adst[:, 0], (0, n_logit - n)).reshape(n_logit // 128, 128)

    sc_gat = _make_sc_kernel(half, nj, c)
    att2, outp0, outp1 = sc_gat(src2, dst2, asrc_p, adst_p, xs)

    out = _tc_combine(outp0, outp1, bias.reshape(1, c))[:n]
    att = att2.reshape(e_pad)[:ep].reshape(ep, 1)
    return (out, (ei, att))


# speedup vs baseline: 30.4560x; 1.3313x over previous
import functools

import jax
import jax.numpy as jnp
from jax import lax
from jax.experimental import pallas as pl
from jax.experimental.pallas import tpu as pltpu
from jax.experimental.pallas import tpu_sc as plsc

NEG_SLOPE = 0.2
LANE = 16
TRASH = 128
PACK = 14


def _prep_body(x_ref, w_ref, asw_ref, adw_ref, xs_ref, asrc_ref, adst_ref, g_ref):
    xs = jnp.dot(x_ref[...], w_ref[...], preferred_element_type=jnp.float32)
    xs_ref[...] = xs
    a_s = jnp.sum(xs * asw_ref[...], axis=1, keepdims=True)
    a_d = jnp.sum(xs * adw_ref[...], axis=1, keepdims=True)
    asrc_ref[...] = a_s
    adst_ref[...] = a_d
    gg = jnp.max(a_s) + jnp.max(a_d)
    g_ref[...] = jnp.full((1, 1), jnp.where(gg >= 0.0, gg, NEG_SLOPE * gg),
                          dtype=jnp.float32)


def _tc_prep(x, w, att_src_row, att_dst_row):
    n = x.shape[0]
    c = w.shape[1]
    return pl.pallas_call(
        _prep_body,
        out_shape=[
            jax.ShapeDtypeStruct((n, c), jnp.float32),
            jax.ShapeDtypeStruct((n, 1), jnp.float32),
            jax.ShapeDtypeStruct((n, 1), jnp.float32),
            jax.ShapeDtypeStruct((1, 1), jnp.float32),
        ],
    )(x, w, att_src_row, att_dst_row)


def _comb_body(p0_ref, p1_ref, b_ref, o_ref):
    o_ref[...] = (jnp.concatenate([p0_ref[...], p1_ref[...]], axis=0)
                  + b_ref[...])


def _tc_combine(p0, p1, brow):
    return pl.pallas_call(
        _comb_body,
        out_shape=jax.ShapeDtypeStruct((2 * p0.shape[0], p0.shape[1]),
                                       jnp.float32),
    )(p0, p1, brow)


def _make_sc_kernel(half, nj, d):
    n_logit = 2 * half + TRASH
    glrow = n_logit // 128
    cap = 2 * nj * 128
    drows = n_logit // 16
    arows = (half + TRASH) // 16
    orows = half // 16
    parts = [(0, nj)] if nj <= 48 else [(0, 48), (48, nj - 48)]
    njb = parts[0][1]
    mesh = plsc.VectorSubcoreMesh(core_axis_name="c", subcore_axis_name="s",
                                  num_cores=2, num_subcores=16)

    @functools.partial(
        pl.kernel,
        out_type=[
            jax.ShapeDtypeStruct((32 * nj, 128), jnp.float32),
            jax.ShapeDtypeStruct((half, d), jnp.float32),
            jax.ShapeDtypeStruct((half, d), jnp.float32),
        ],
        mesh=mesh,
        scratch_types=[
            pltpu.VMEM((glrow + 1, 128), jnp.float32),
            pltpu.VMEM((glrow, 128), jnp.float32),
            pltpu.VMEM((njb, 128), jnp.int32),
            pltpu.VMEM((njb, 128), jnp.int32),
            pltpu.VMEM((1, 128), jnp.float32),
            pltpu.VMEM((8, 128), jnp.float32),
            pltpu.VMEM((cap,), jnp.int32),
            pltpu.VMEM((n_logit,), jnp.float32),
            pltpu.VMEM((64, d), jnp.float32),
            pltpu.VMEM((64, d), jnp.float32),
            pltpu.VMEM((64,), jnp.float32),
            pltpu.VMEM((1, 64), jnp.int32),
            pltpu.VMEM((1, 64), jnp.int32),
            pltpu.VMEM((64,), jnp.int32),
            pltpu.VMEM((64,), jnp.int32),
            pltpu.VMEM_SHARED((n_logit,), jnp.float32),
            pltpu.VMEM_SHARED((half + TRASH, d), jnp.float32),
            pltpu.SemaphoreType.DMA,
            pltpu.SemaphoreType.DMA,
            pltpu.SemaphoreType.DMA,
            pltpu.SemaphoreType.DMA,
        ],
        compiler_params=pltpu.CompilerParams(needs_layout_passes=False),
    )
    def sc_gat(src_hbm, dst_hbm, asrc_hbm, adst_hbm, xs_hbm,
               att_hbm, outp0_hbm, outp1_hbm, asrc_v, adst_v, src_v, dst_v,
               prow_v, att8_v, packedC, denom_v, rows_a, rows_b, attw_v,
               dstRow_a, dstRow_b, srcRow_a, srcRow_b, denom_sh, out_sh,
               gsem_a, gsem_b, ssem_a, ssem_b):
        c = lax.axis_index("c")
        s = lax.axis_index("s")
        zv = jnp.zeros((LANE,), jnp.float32)

        def body_zd(i, carry):
            denom_v[pl.ds(i * LANE, LANE)] = zv
            return carry

        lax.fori_loop(0, n_logit // LANE, body_zd, 0)

        def body_zr(r, carry):
            for i in range(d // LANE):
                rows_a[r, pl.ds(i * LANE, LANE)] = zv
            return carry

        lax.fori_loop(0, 64, body_zr, 0)
        pltpu.sync_copy(denom_v.at[pl.ds(s * drows, drows)],
                        denom_sh.at[pl.ds(s * drows, drows)])
        off = 0
        while off < arows:
            m = min(64, arows - off)
            pltpu.sync_copy(rows_a.at[pl.ds(0, m)],
                            out_sh.at[pl.ds(s * arows + off, m)])
            off += m
        pltpu.sync_copy(asrc_hbm, asrc_v)
        pltpu.sync_copy(adst_hbm, adst_v)
        plsc.subcore_barrier()
        g_vec = asrc_v[glrow, pl.ds(0, LANE)]

        def edge_p(sv, dv):
            a = plsc.load_gather(asrc_v, [sv >> 7, sv & 127])
            b = plsc.load_gather(adst_v, [dv >> 7, dv & 127])
            al = a + b
            al = jnp.where(al >= 0.0, al, al * NEG_SLOPE)
            return jnp.exp(al - g_vec)

        cnt = jnp.int32(0)
        for which in range(2):
            gchunk = 2 * s + (1 - c) if which == 0 else 2 * s + c
            for off_r, rr in parts:
                pltpu.sync_copy(src_hbm.at[pl.ds(gchunk * nj + off_r, rr)],
                                src_v.at[pl.ds(0, rr)])
                pltpu.sync_copy(dst_hbm.at[pl.ds(gchunk * nj + off_r, rr)],
                                dst_v.at[pl.ds(0, rr)])

                def body_j(j, cnt):
                    for i in range(128 // LANE):
                        sl = pl.ds(i * LANE, LANE)
                        sv = src_v[j, sl]
                        dv = dst_v[j, sl]
                        prow_v[0, sl] = edge_p(sv, dv)
                        dvr = dv - c * half
                        keep = (dvr >= 0) & (dvr < half)
                        plsc.store_compressed(packedC.at[pl.ds(cnt, LANE)],
                                              sv | (dvr << PACK), mask=keep)
                        cnt = cnt + plsc.all_reduce_population_count(keep)[0]
                    pltpu.sync_copy(prow_v.at[0], denom_sh.at[dst_v.at[j]],
                                    add=True)
                    return cnt

                cnt = lax.fori_loop(0, rr, body_j, cnt)

        iota16 = lax.iota(jnp.int32, LANE)
        rem = (64 - (cnt & 63)) & 63

        def body_pad(i, carry):
            cnt, rem = carry
            t = jnp.minimum(rem, LANE)
            plsc.store_compressed(packedC.at[pl.ds(cnt, LANE)],
                                  (half + iota16) << PACK,
                                  mask=iota16 < t)
            return (cnt + t, rem - t)

        cnt, _ = lax.fori_loop(0, 4, body_pad, (cnt, rem))
        ngrp = cnt >> 6
        plsc.subcore_barrier()

        pltpu.sync_copy(denom_sh, denom_v)

        def body_rcp(i, carry):
            sl = pl.ds(i * LANE, LANE)
            denom_v[sl] = 1.0 / denom_v[sl]
            return carry

        lax.fori_loop(0, n_logit // LANE, body_rcp, 0)

        for off_r, rr in parts:
            pltpu.sync_copy(
                src_hbm.at[pl.ds((2 * s + c) * nj + off_r, rr)],
                src_v.at[pl.ds(0, rr)])
            pltpu.sync_copy(
                dst_hbm.at[pl.ds((2 * s + c) * nj + off_r, rr)],
                dst_v.at[pl.ds(0, rr)])

            def body_jo(jo, carry):
                for jj in range(8):
                    j = jo * 8 + jj
                    for i in range(128 // LANE):
                        sl = pl.ds(i * LANE, LANE)
                        sv = src_v[j, sl]
                        dv = dst_v[j, sl]
                        r = plsc.load_gather(denom_v, [dv])
                        att8_v[jj, sl] = edge_p(sv, dv) * r
                pltpu.sync_copy(
                    att8_v,
                    att_hbm.at[pl.ds((2 * s + c) * nj + off_r + jo * 8, 8)])
                return carry

            lax.fori_loop(0, rr // 8, body_jo, 0)

        G = 64
        bufs = [rows_a, rows_b]
        srows = [srcRow_a, srcRow_b]
        drows_i = [dstRow_a, dstRow_b]
        gsems = [gsem_a, gsem_b]
        ssems = [ssem_a, ssem_b]

        def unpack_idx(g2, b):
            base = g2 * G
            for i in range(G // LANE):
                sl = pl.ds(i * LANE, LANE)
                pk = packedC[pl.ds(base + i * LANE, LANE)]
                srows[b][sl] = pk & ((1 << PACK) - 1)
                drows_i[b][0, sl] = pk >> PACK

        def wait_gather(b):
            pltpu.make_async_copy(xs_hbm.at[srows[b]], bufs[b],
                                  gsems[b]).wait()

        def wait_scatter(b):
            pltpu.make_async_copy(bufs[b], out_sh.at[drows_i[b].at[0]],
                                  ssems[b]).wait()

        def compute_group(g2, b):
            base = g2 * G
            for i in range(G // LANE):
                sl = pl.ds(i * LANE, LANE)
                pk = packedC[pl.ds(base + i * LANE, LANE)]
                sC = pk & ((1 << PACK) - 1)
                dvr = pk >> PACK
                dabs = dvr + c * half
                r = plsc.load_gather(denom_v, [dabs])
                attw_v[sl] = edge_p(sC, dabs) * r

            def body_e(e, ecarry):
                ab = plsc.load_gather(attw_v,
                                      [jnp.full((LANE,), e, jnp.int32)])
                for f in range(d // LANE):
                    slf = pl.ds(f * LANE, LANE)
                    bufs[b][e, slf] = bufs[b][e, slf] * ab
                return ecarry

            lax.fori_loop(0, G, body_e, 0)

        @pl.when(ngrp > 0)
        def _():
            unpack_idx(0, 0)
            pltpu.async_copy(xs_hbm.at[srcRow_a], rows_a, gsem_a)

        def body_pair(t, carry):
            for b in range(2):
                g2 = t * 2 + b
                ob = 1 - b

                @pl.when(g2 < ngrp)
                def _(g2=g2, b=b, ob=ob):
                    wait_gather(b)

                    @pl.when(g2 + 1 < ngrp)
                    def _(g2=g2, ob=ob):
                        unpack_idx(g2 + 1, ob)

                        @pl.when(g2 >= 1)
                        def _(ob=ob):
                            wait_scatter(ob)

                        pltpu.async_copy(xs_hbm.at[srows[ob]], bufs[ob],
                                         gsems[ob])

                    compute_group(g2, b)
                    pltpu.async_copy(bufs[b], out_sh.at[drows_i[b].at[0]],
                                     ssems[b], add=True)
            return carry

        lax.fori_loop(0, (ngrp + 1) >> 1, body_pair, 0)

        @pl.when(ngrp >= 1)
        def _():
            wait_scatter(0)

        @pl.when(ngrp >= 2)
        def _():
            wait_scatter(1)

        plsc.subcore_barrier()

        off = 0
        while off < orows:
            m = min(64, orows - off)
            pltpu.sync_copy(out_sh.at[pl.ds(s * orows + off, m)],
                            rows_a.at[pl.ds(0, m)])

            @pl.when(c == 0)
            def _(off=off, m=m):
                pltpu.sync_copy(rows_a.at[pl.ds(0, m)],
                                outp0_hbm.at[pl.ds(s * orows + off, m)])

            @pl.when(c == 1)
            def _(off=off, m=m):
                pltpu.sync_copy(rows_a.at[pl.ds(0, m)],
                                outp1_hbm.at[pl.ds(s * orows + off, m)])

            off += m

    return sc_gat


def kernel(x, edge_index, W, att_src, att_dst, bias):
    n, din = x.shape
    c = W.shape[1]
    e = edge_index.shape[1]
    ep = e + n
    nj = -(-ep // (32 * 128))
    nj = -(-nj // 8) * 8
    e_pad = 32 * nj * 128
    half = -(-(n // 2 + 1) // 128) * 128
    if 2 * half <= n:
        half += 128
    n_acc = 2 * half

    loops = jnp.arange(n, dtype=edge_index.dtype)
    ei = jnp.concatenate([edge_index, jnp.stack([loops, loops])], axis=1)
    pad = e_pad - ep
    src = jnp.concatenate([ei[0], jnp.zeros((pad,), jnp.int32)])
    pad_dst = n_acc + (jnp.arange(pad, dtype=jnp.int32) % TRASH)
    dst = jnp.concatenate([ei[1], pad_dst])
    src2 = src.reshape(e_pad // 128, 128)
    dst2 = dst.reshape(e_pad // 128, 128)

    xs, asrc, adst, g = _tc_prep(x, W, att_src.reshape(1, c),
                                 att_dst.reshape(1, c))
    n_logit = n_acc + TRASH
    asrc_p = jnp.concatenate(
        [jnp.pad(asrc[:, 0], (0, n_logit - n)),
         jnp.broadcast_to(g.reshape(1), (128,))]).reshape(
             n_logit // 128 + 1, 128)
    adst_p = jnp.pad(adst[:, 0], (0, n_logit - n)).reshape(n_logit // 128, 128)

    sc_gat = _make_sc_kernel(half, nj, c)
    att2, outp0, outp1 = sc_gat(src2, dst2, asrc_p, adst_p, xs)

    out = _tc_combine(outp0, outp1, bias.reshape(1, c))[:n]
    att = att2.reshape(e_pad)[:ep].reshape(ep, 1)
    return (out, (ei, att))


# speedup vs baseline: 31.2001x; 1.0244x over previous
import functools

import jax
import jax.numpy as jnp
from jax import lax
from jax.experimental import pallas as pl
from jax.experimental.pallas import tpu as pltpu
from jax.experimental.pallas import tpu_sc as plsc

NEG_SLOPE = 0.2
LANE = 16
TRASH = 128
PACK = 14


def _prep_body(x_ref, w_ref, asw_ref, adw_ref, xs_ref, asrc_ref, adst_ref, g_ref):
    xs = jnp.dot(x_ref[...], w_ref[...], preferred_element_type=jnp.float32)
    xs_ref[...] = xs
    a_s = jnp.sum(xs * asw_ref[...], axis=1, keepdims=True)
    a_d = jnp.sum(xs * adw_ref[...], axis=1, keepdims=True)
    asrc_ref[...] = a_s
    adst_ref[...] = a_d
    gg = jnp.max(a_s) + jnp.max(a_d)
    g_ref[...] = jnp.full((1, 1), jnp.where(gg >= 0.0, gg, NEG_SLOPE * gg),
                          dtype=jnp.float32)


def _tc_prep(x, w, att_src_row, att_dst_row):
    n = x.shape[0]
    c = w.shape[1]
    return pl.pallas_call(
        _prep_body,
        out_shape=[
            jax.ShapeDtypeStruct((n, c), jnp.float32),
            jax.ShapeDtypeStruct((n, 1), jnp.float32),
            jax.ShapeDtypeStruct((n, 1), jnp.float32),
            jax.ShapeDtypeStruct((1, 1), jnp.float32),
        ],
    )(x, w, att_src_row, att_dst_row)


def _comb_body(p0_ref, p1_ref, b_ref, o_ref):
    o_ref[...] = (jnp.concatenate([p0_ref[...], p1_ref[...]], axis=0)
                  + b_ref[...])


def _tc_combine(p0, p1, brow):
    return pl.pallas_call(
        _comb_body,
        out_shape=jax.ShapeDtypeStruct((2 * p0.shape[0], p0.shape[1]),
                                       jnp.float32),
    )(p0, p1, brow)


def _make_sc_kernel(half, nj, d):
    n_logit = 2 * half + TRASH
    glrow = n_logit // 128
    cap = 2 * nj * 128
    drows = n_logit // 16
    arows = (half + TRASH) // 16
    orows = half // 16
    parts = [(0, nj)] if nj <= 48 else [(0, 48), (48, nj - 48)]
    njb = parts[0][1]
    mesh = plsc.VectorSubcoreMesh(core_axis_name="c", subcore_axis_name="s",
                                  num_cores=2, num_subcores=16)

    @functools.partial(
        pl.kernel,
        out_type=[
            jax.ShapeDtypeStruct((32 * nj, 128), jnp.float32),
            jax.ShapeDtypeStruct((half, d), jnp.float32),
            jax.ShapeDtypeStruct((half, d), jnp.float32),
        ],
        mesh=mesh,
        scratch_types=[
            pltpu.VMEM((glrow + 1, 128), jnp.float32),
            pltpu.VMEM((glrow, 128), jnp.float32),
            pltpu.VMEM((njb, 128), jnp.int32),
            pltpu.VMEM((njb, 128), jnp.int32),
            pltpu.VMEM((1, 128), jnp.float32),
            pltpu.VMEM((8, 128), jnp.float32),
            pltpu.VMEM((cap,), jnp.int32),
            pltpu.VMEM((n_logit,), jnp.float32),
            pltpu.VMEM((64, d), jnp.float32),
            pltpu.VMEM((64, d), jnp.float32),
            pltpu.VMEM((64,), jnp.float32),
            pltpu.VMEM((1, 64), jnp.int32),
            pltpu.VMEM((1, 64), jnp.int32),
            pltpu.VMEM((64,), jnp.int32),
            pltpu.VMEM((64,), jnp.int32),
            pltpu.VMEM_SHARED((n_logit,), jnp.float32),
            pltpu.VMEM_SHARED((half + TRASH, d), jnp.float32),
            pltpu.SemaphoreType.DMA,
            pltpu.SemaphoreType.DMA,
            pltpu.SemaphoreType.DMA,
            pltpu.SemaphoreType.DMA,
        ],
        compiler_params=pltpu.CompilerParams(needs_layout_passes=False),
    )
    def sc_gat(src_hbm, dst_hbm, asrc_hbm, adst_hbm, xs_hbm,
               att_hbm, outp0_hbm, outp1_hbm, asrc_v, adst_v, src_v, dst_v,
               prow_v, att8_v, packedC, denom_v, rows_a, rows_b, attw_v,
               dstRow_a, dstRow_b, srcRow_a, srcRow_b, denom_sh, out_sh,
               gsem_a, gsem_b, ssem_a, ssem_b):
        c = lax.axis_index("c")
        s = lax.axis_index("s")
        zv = jnp.zeros((LANE,), jnp.float32)

        def body_zd(i, carry):
            denom_v[pl.ds(i * LANE, LANE)] = zv
            return carry

        lax.fori_loop(0, n_logit // LANE, body_zd, 0)

        def body_zr(r, carry):
            for i in range(d // LANE):
                rows_a[r, pl.ds(i * LANE, LANE)] = zv
            return carry

        lax.fori_loop(0, 64, body_zr, 0)
        pltpu.sync_copy(denom_v.at[pl.ds(s * drows, drows)],
                        denom_sh.at[pl.ds(s * drows, drows)])
        off = 0
        while off < arows:
            m = min(64, arows - off)
            pltpu.sync_copy(rows_a.at[pl.ds(0, m)],
                            out_sh.at[pl.ds(s * arows + off, m)])
            off += m
        pltpu.sync_copy(asrc_hbm, asrc_v)
        pltpu.sync_copy(adst_hbm, adst_v)
        plsc.subcore_barrier()
        g_vec = asrc_v[glrow, pl.ds(0, LANE)]

        def edge_p(sv, dv):
            a = plsc.load_gather(asrc_v, [sv >> 7, sv & 127])
            b = plsc.load_gather(adst_v, [dv >> 7, dv & 127])
            al = a + b
            al = jnp.where(al >= 0.0, al, al * NEG_SLOPE)
            return jnp.exp(al - g_vec)

        cnt = jnp.int32(0)
        for which in range(2):
            gchunk = 2 * s + (1 - c) if which == 0 else 2 * s + c
            for off_r, rr in parts:
                pltpu.sync_copy(src_hbm.at[pl.ds(gchunk * nj + off_r, rr)],
                                src_v.at[pl.ds(0, rr)])
                pltpu.sync_copy(dst_hbm.at[pl.ds(gchunk * nj + off_r, rr)],
                                dst_v.at[pl.ds(0, rr)])

                def body_jo(jo, cnt):
                    for jj in range(8):
                        j = jo * 8 + jj
                        for i in range(128 // LANE):
                            sl = pl.ds(i * LANE, LANE)
                            sv = src_v[j, sl]
                            dv = dst_v[j, sl]
                            att8_v[jj, sl] = edge_p(sv, dv)
                            dvr = dv - c * half
                            keep = (dvr >= 0) & (dvr < half)
                            plsc.store_compressed(
                                packedC.at[pl.ds(cnt, LANE)],
                                sv | (dvr << PACK), mask=keep)
                            cnt = cnt + \
                                plsc.all_reduce_population_count(keep)[0]
                        pltpu.async_copy(att8_v.at[jj],
                                         denom_sh.at[dst_v.at[j]],
                                         gsem_a, add=True)
                    for jj in range(8):
                        pltpu.make_async_copy(
                            att8_v.at[jj],
                            denom_sh.at[dst_v.at[jo * 8 + jj]],
                            gsem_a).wait()
                    return cnt

                cnt = lax.fori_loop(0, rr // 8, body_jo, cnt)

        iota16 = lax.iota(jnp.int32, LANE)
        rem = (64 - (cnt & 63)) & 63

        def body_pad(i, carry):
            cnt, rem = carry
            t = jnp.minimum(rem, LANE)
            plsc.store_compressed(packedC.at[pl.ds(cnt, LANE)],
                                  (half + iota16) << PACK,
                                  mask=iota16 < t)
            return (cnt + t, rem - t)

        cnt, _ = lax.fori_loop(0, 4, body_pad, (cnt, rem))
        ngrp = cnt >> 6
        plsc.subcore_barrier()

        pltpu.sync_copy(denom_sh, denom_v)

        def body_rcp(i, carry):
            sl = pl.ds(i * LANE, LANE)
            denom_v[sl] = 1.0 / denom_v[sl]
            return carry

        lax.fori_loop(0, n_logit // LANE, body_rcp, 0)

        for off_r, rr in parts:
            pltpu.sync_copy(
                src_hbm.at[pl.ds((2 * s + c) * nj + off_r, rr)],
                src_v.at[pl.ds(0, rr)])
            pltpu.sync_copy(
                dst_hbm.at[pl.ds((2 * s + c) * nj + off_r, rr)],
                dst_v.at[pl.ds(0, rr)])

            def body_jo(jo, carry):
                for jj in range(8):
                    j = jo * 8 + jj
                    for i in range(128 // LANE):
                        sl = pl.ds(i * LANE, LANE)
                        sv = src_v[j, sl]
                        dv = dst_v[j, sl]
                        r = plsc.load_gather(denom_v, [dv])
                        att8_v[jj, sl] = edge_p(sv, dv) * r
                pltpu.sync_copy(
                    att8_v,
                    att_hbm.at[pl.ds((2 * s + c) * nj + off_r + jo * 8, 8)])
                return carry

            lax.fori_loop(0, rr // 8, body_jo, 0)

        G = 64
        bufs = [rows_a, rows_b]
        srows = [srcRow_a, srcRow_b]
        drows_i = [dstRow_a, dstRow_b]
        gsems = [gsem_a, gsem_b]
        ssems = [ssem_a, ssem_b]

        def unpack_idx(g2, b):
            base = g2 * G
            for i in range(G // LANE):
                sl = pl.ds(i * LANE, LANE)
                pk = packedC[pl.ds(base + i * LANE, LANE)]
                srows[b][sl] = pk & ((1 << PACK) - 1)
                drows_i[b][0, sl] = pk >> PACK

        def wait_gather(b):
            pltpu.make_async_copy(xs_hbm.at[srows[b]], bufs[b],
                                  gsems[b]).wait()

        def wait_scatter(b):
            pltpu.make_async_copy(bufs[b], out_sh.at[drows_i[b].at[0]],
                                  ssems[b]).wait()

        def compute_group(g2, b):
            base = g2 * G
            for i in range(G // LANE):
                sl = pl.ds(i * LANE, LANE)
                pk = packedC[pl.ds(base + i * LANE, LANE)]
                sC = pk & ((1 << PACK) - 1)
                dvr = pk >> PACK
                dabs = dvr + c * half
                r = plsc.load_gather(denom_v, [dabs])
                attw_v[sl] = edge_p(sC, dabs) * r

            def body_e(e, ecarry):
                ab = plsc.load_gather(attw_v,
                                      [jnp.full((LANE,), e, jnp.int32)])
                for f in range(d // LANE):
                    slf = pl.ds(f * LANE, LANE)
                    bufs[b][e, slf] = bufs[b][e, slf] * ab
                return ecarry

            lax.fori_loop(0, G, body_e, 0)

        @pl.when(ngrp > 0)
        def _():
            unpack_idx(0, 0)
            pltpu.async_copy(xs_hbm.at[srcRow_a], rows_a, gsem_a)

        def body_pair(t, carry):
            for b in range(2):
                g2 = t * 2 + b
                ob = 1 - b

                @pl.when(g2 < ngrp)
                def _(g2=g2, b=b, ob=ob):
                    wait_gather(b)

                    @pl.when(g2 + 1 < ngrp)
                    def _(g2=g2, ob=ob):
                        unpack_idx(g2 + 1, ob)

                        @pl.when(g2 >= 1)
                        def _(ob=ob):
                            wait_scatter(ob)

                        pltpu.async_copy(xs_hbm.at[srows[ob]], bufs[ob],
                                         gsems[ob])

                    compute_group(g2, b)
                    pltpu.async_copy(bufs[b], out_sh.at[drows_i[b].at[0]],
                                     ssems[b], add=True)
            return carry

        lax.fori_loop(0, (ngrp + 1) >> 1, body_pair, 0)

        @pl.when(ngrp >= 1)
        def _():
            wait_scatter(0)

        @pl.when(ngrp >= 2)
        def _():
            wait_scatter(1)

        plsc.subcore_barrier()

        off = 0
        while off < orows:
            m = min(64, orows - off)
            pltpu.sync_copy(out_sh.at[pl.ds(s * orows + off, m)],
                            rows_a.at[pl.ds(0, m)])

            @pl.when(c == 0)
            def _(off=off, m=m):
                pltpu.sync_copy(rows_a.at[pl.ds(0, m)],
                                outp0_hbm.at[pl.ds(s * orows + off, m)])

            @pl.when(c == 1)
            def _(off=off, m=m):
                pltpu.sync_copy(rows_a.at[pl.ds(0, m)],
                                outp1_hbm.at[pl.ds(s * orows + off, m)])

            off += m

    return sc_gat


def kernel(x, edge_index, W, att_src, att_dst, bias):
    n, din = x.shape
    c = W.shape[1]
    e = edge_index.shape[1]
    ep = e + n
    nj = -(-ep // (32 * 128))
    nj = -(-nj // 8) * 8
    e_pad = 32 * nj * 128
    half = -(-(n // 2 + 1) // 128) * 128
    if 2 * half <= n:
        half += 128
    n_acc = 2 * half

    loops = jnp.arange(n, dtype=edge_index.dtype)
    ei = jnp.concatenate([edge_index, jnp.stack([loops, loops])], axis=1)
    pad = e_pad - ep
    src = jnp.concatenate([ei[0], jnp.zeros((pad,), jnp.int32)])
    pad_dst = n_acc + (jnp.arange(pad, dtype=jnp.int32) % TRASH)
    dst = jnp.concatenate([ei[1], pad_dst])
    src2 = src.reshape(e_pad // 128, 128)
    dst2 = dst.reshape(e_pad // 128, 128)

    xs, asrc, adst, g = _tc_prep(x, W, att_src.reshape(1, c),
                                 att_dst.reshape(1, c))
    n_logit = n_acc + TRASH
    asrc_p = jnp.concatenate(
        [jnp.pad(asrc[:, 0], (0, n_logit - n)),
         jnp.broadcast_to(g.reshape(1), (128,))]).reshape(
             n_logit // 128 + 1, 128)
    adst_p = jnp.pad(adst[:, 0], (0, n_logit - n)).reshape(n_logit // 128, 128)

    sc_gat = _make_sc_kernel(half, nj, c)
    att2, outp0, outp1 = sc_gat(src2, dst2, asrc_p, adst_p, xs)

    out = _tc_combine(outp0, outp1, bias.reshape(1, c))[:n]
    att = att2.reshape(e_pad)[:ep].reshape(ep, 1)
    return (out, (ei, att))


# speedup vs baseline: 31.9863x; 1.0252x over previous
import functools

import jax
import jax.numpy as jnp
from jax import lax
from jax.experimental import pallas as pl
from jax.experimental.pallas import tpu as pltpu
from jax.experimental.pallas import tpu_sc as plsc

NEG_SLOPE = 0.2
LANE = 16
TRASH = 128
PACK = 14


def _prep_body(x_ref, w_ref, asw_ref, adw_ref, xs_ref, asrc_ref, adst_ref, g_ref):
    xs = jnp.dot(x_ref[...], w_ref[...], preferred_element_type=jnp.float32)
    xs_ref[...] = xs
    a_s = jnp.sum(xs * asw_ref[...], axis=1, keepdims=True)
    a_d = jnp.sum(xs * adw_ref[...], axis=1, keepdims=True)
    asrc_ref[...] = a_s
    adst_ref[...] = a_d
    gg = jnp.max(a_s) + jnp.max(a_d)
    g_ref[...] = jnp.full((1, 1), jnp.where(gg >= 0.0, gg, NEG_SLOPE * gg),
                          dtype=jnp.float32)


def _tc_prep(x, w, att_src_row, att_dst_row):
    n = x.shape[0]
    c = w.shape[1]
    return pl.pallas_call(
        _prep_body,
        out_shape=[
            jax.ShapeDtypeStruct((n, c), jnp.float32),
            jax.ShapeDtypeStruct((n, 1), jnp.float32),
            jax.ShapeDtypeStruct((n, 1), jnp.float32),
            jax.ShapeDtypeStruct((1, 1), jnp.float32),
        ],
    )(x, w, att_src_row, att_dst_row)


def _comb_body(p0_ref, p1_ref, b_ref, o_ref):
    o_ref[...] = (jnp.concatenate([p0_ref[...], p1_ref[...]], axis=0)
                  + b_ref[...])


def _tc_combine(p0, p1, brow):
    return pl.pallas_call(
        _comb_body,
        out_shape=jax.ShapeDtypeStruct((2 * p0.shape[0], p0.shape[1]),
                                       jnp.float32),
    )(p0, p1, brow)


def _make_sc_kernel(half, nj, d):
    n_logit = 2 * half + TRASH
    glrow = n_logit // 128
    cap = 2 * nj * 128
    drows = n_logit // 16
    arows = (half + TRASH) // 16
    orows = half // 16
    parts = [(0, nj)] if nj <= 48 else [(0, 48), (48, nj - 48)]
    njb = parts[0][1]
    mesh = plsc.VectorSubcoreMesh(core_axis_name="c", subcore_axis_name="s",
                                  num_cores=2, num_subcores=16)

    @functools.partial(
        pl.kernel,
        out_type=[
            jax.ShapeDtypeStruct((32 * nj, 128), jnp.float32),
            jax.ShapeDtypeStruct((half, d), jnp.float32),
            jax.ShapeDtypeStruct((half, d), jnp.float32),
        ],
        mesh=mesh,
        scratch_types=[
            pltpu.VMEM((glrow + 1, 128), jnp.float32),
            pltpu.VMEM((glrow, 128), jnp.float32),
            pltpu.VMEM((njb, 128), jnp.int32),
            pltpu.VMEM((njb, 128), jnp.int32),
            pltpu.VMEM((1, 128), jnp.float32),
            pltpu.VMEM((8, 128), jnp.float32),
            pltpu.VMEM((cap,), jnp.int32),
            pltpu.VMEM((n_logit,), jnp.float32),
            pltpu.VMEM((64, d), jnp.float32),
            pltpu.VMEM((64, d), jnp.float32),
            pltpu.VMEM((64,), jnp.float32),
            pltpu.VMEM((1, 64), jnp.int32),
            pltpu.VMEM((1, 64), jnp.int32),
            pltpu.VMEM((64,), jnp.int32),
            pltpu.VMEM((64,), jnp.int32),
            pltpu.VMEM_SHARED((n_logit,), jnp.float32),
            pltpu.VMEM_SHARED((half + TRASH, d), jnp.float32),
            pltpu.SemaphoreType.DMA,
            pltpu.SemaphoreType.DMA,
            pltpu.SemaphoreType.DMA,
            pltpu.SemaphoreType.DMA,
        ],
        compiler_params=pltpu.CompilerParams(needs_layout_passes=False),
    )
    def sc_gat(src_hbm, dst_hbm, asrc_hbm, adst_hbm, xs_hbm,
               att_hbm, outp0_hbm, outp1_hbm, asrc_v, adst_v, src_v, dst_v,
               prow_v, att8_v, packedC, denom_v, rows_a, rows_b, attw_v,
               dstRow_a, dstRow_b, srcRow_a, srcRow_b, denom_sh, out_sh,
               gsem_a, gsem_b, ssem_a, ssem_b):
        c = lax.axis_index("c")
        s = lax.axis_index("s")
        zv = jnp.zeros((LANE,), jnp.float32)

        def body_zd(i, carry):
            denom_v[pl.ds(i * LANE, LANE)] = zv
            return carry

        lax.fori_loop(0, n_logit // LANE, body_zd, 0)

        def body_zr(r, carry):
            for i in range(d // LANE):
                rows_a[r, pl.ds(i * LANE, LANE)] = zv
            return carry

        lax.fori_loop(0, 64, body_zr, 0)
        pltpu.sync_copy(denom_v.at[pl.ds(s * drows, drows)],
                        denom_sh.at[pl.ds(s * drows, drows)])
        off = 0
        while off < arows:
            m = min(64, arows - off)
            pltpu.sync_copy(rows_a.at[pl.ds(0, m)],
                            out_sh.at[pl.ds(s * arows + off, m)])
            off += m
        pltpu.sync_copy(asrc_hbm, asrc_v)
        pltpu.sync_copy(adst_hbm, adst_v)
        plsc.subcore_barrier()
        g_vec = asrc_v[glrow, pl.ds(0, LANE)]

        def edge_p(sv, dv):
            a = plsc.load_gather(asrc_v, [sv >> 7, sv & 127])
            b = plsc.load_gather(adst_v, [dv >> 7, dv & 127])
            al = a + b
            al = jnp.where(al >= 0.0, al, al * NEG_SLOPE)
            return jnp.exp(al - g_vec)

        cnt = jnp.int32(0)
        for which in range(2):
            gchunk = 2 * s + (1 - c) if which == 0 else 2 * s + c
            for off_r, rr in parts:
                pltpu.sync_copy(src_hbm.at[pl.ds(gchunk * nj + off_r, rr)],
                                src_v.at[pl.ds(0, rr)])
                pltpu.sync_copy(dst_hbm.at[pl.ds(gchunk * nj + off_r, rr)],
                                dst_v.at[pl.ds(0, rr)])

                def body_jo(jo, cnt):
                    for jj in range(8):
                        j = jo * 8 + jj
                        for i in range(128 // LANE):
                            sl = pl.ds(i * LANE, LANE)
                            sv = src_v[j, sl]
                            dv = dst_v[j, sl]
                            att8_v[jj, sl] = edge_p(sv, dv)
                            dvr = dv - c * half
                            keep = (dvr >= 0) & (dvr < half)
                            plsc.store_compressed(
                                packedC.at[pl.ds(cnt, LANE)],
                                sv | (dvr << PACK), mask=keep)
                            cnt = cnt + \
                                plsc.all_reduce_population_count(keep)[0]
                        pltpu.async_copy(att8_v.at[jj],
                                         denom_sh.at[dst_v.at[j]],
                                         gsem_a, add=True)
                    for jj in range(8):
                        pltpu.make_async_copy(
                            att8_v.at[jj],
                            denom_sh.at[dst_v.at[jo * 8 + jj]],
                            gsem_a).wait()
                    return cnt

                cnt = lax.fori_loop(0, rr // 8, body_jo, cnt)

        iota16 = lax.iota(jnp.int32, LANE)
        rem = (64 - (cnt & 63)) & 63

        def body_pad(i, carry):
            cnt, rem = carry
            t = jnp.minimum(rem, LANE)
            plsc.store_compressed(packedC.at[pl.ds(cnt, LANE)],
                                  (half + iota16) << PACK,
                                  mask=iota16 < t)
            return (cnt + t, rem - t)

        cnt, _ = lax.fori_loop(0, 4, body_pad, (cnt, rem))
        ngrp = cnt >> 6
        plsc.subcore_barrier()

        pltpu.sync_copy(denom_sh, denom_v)

        def body_rcp(i, carry):
            sl = pl.ds(i * LANE, LANE)
            denom_v[sl] = 1.0 / denom_v[sl]
            return carry

        lax.fori_loop(0, n_logit // LANE, body_rcp, 0)

        for off_r, rr in parts:
            pltpu.sync_copy(
                src_hbm.at[pl.ds((2 * s + c) * nj + off_r, rr)],
                src_v.at[pl.ds(0, rr)])
            pltpu.sync_copy(
                dst_hbm.at[pl.ds((2 * s + c) * nj + off_r, rr)],
                dst_v.at[pl.ds(0, rr)])

            def body_jo(jo, carry):
                for jj in range(8):
                    j = jo * 8 + jj
                    for i in range(128 // LANE):
                        sl = pl.ds(i * LANE, LANE)
                        sv = src_v[j, sl]
                        dv = dst_v[j, sl]
                        r = plsc.load_gather(denom_v, [dv])
                        att8_v[jj, sl] = edge_p(sv, dv) * r
                pltpu.sync_copy(
                    att8_v,
                    att_hbm.at[pl.ds((2 * s + c) * nj + off_r + jo * 8, 8)])
                return carry

            lax.fori_loop(0, rr // 8, body_jo, 0)

        G = 64
        bufs = [rows_a, rows_b]
        srows = [srcRow_a, srcRow_b]
        drows_i = [dstRow_a, dstRow_b]
        gsems = [gsem_a, gsem_b]
        ssems = [ssem_a, ssem_b]

        def unpack_idx(g2, b):
            base = g2 * G
            for i in range(G // LANE):
                sl = pl.ds(i * LANE, LANE)
                pk = packedC[pl.ds(base + i * LANE, LANE)]
                srows[b][sl] = pk & ((1 << PACK) - 1)
                drows_i[b][0, sl] = pk >> PACK

        def wait_gather(b):
            pltpu.make_async_copy(xs_hbm.at[srows[b]], bufs[b],
                                  gsems[b]).wait()

        def wait_scatter(b):
            pltpu.make_async_copy(bufs[b], out_sh.at[drows_i[b].at[0]],
                                  ssems[b]).wait()

        def compute_group(g2, b):
            base = g2 * G
            for i in range(G // LANE):
                sl = pl.ds(i * LANE, LANE)
                pk = packedC[pl.ds(base + i * LANE, LANE)]
                sC = pk & ((1 << PACK) - 1)
                dvr = pk >> PACK
                dabs = dvr + c * half
                r = plsc.load_gather(denom_v, [dabs])
                attw_v[sl] = edge_p(sC, dabs) * r

            def body_e(e2, ecarry):
                for u in range(4):
                    e = e2 * 4 + u
                    ab = plsc.load_gather(attw_v,
                                          [jnp.full((LANE,), e, jnp.int32)])
                    for f in range(d // LANE):
                        slf = pl.ds(f * LANE, LANE)
                        bufs[b][e, slf] = bufs[b][e, slf] * ab
                return ecarry

            lax.fori_loop(0, G // 4, body_e, 0)

        @pl.when(ngrp > 0)
        def _():
            unpack_idx(0, 0)
            pltpu.async_copy(xs_hbm.at[srcRow_a], rows_a, gsem_a)

        def body_pair(t, carry):
            for b in range(2):
                g2 = t * 2 + b
                ob = 1 - b

                @pl.when(g2 < ngrp)
                def _(g2=g2, b=b, ob=ob):
                    wait_gather(b)

                    @pl.when(g2 + 1 < ngrp)
                    def _(g2=g2, ob=ob):
                        unpack_idx(g2 + 1, ob)

                        @pl.when(g2 >= 1)
                        def _(ob=ob):
                            wait_scatter(ob)

                        pltpu.async_copy(xs_hbm.at[srows[ob]], bufs[ob],
                                         gsems[ob])

                    compute_group(g2, b)
                    pltpu.async_copy(bufs[b], out_sh.at[drows_i[b].at[0]],
                                     ssems[b], add=True)
            return carry

        lax.fori_loop(0, (ngrp + 1) >> 1, body_pair, 0)

        @pl.when(ngrp >= 1)
        def _():
            wait_scatter(0)

        @pl.when(ngrp >= 2)
        def _():
            wait_scatter(1)

        plsc.subcore_barrier()

        off = 0
        while off < orows:
            m = min(64, orows - off)
            pltpu.sync_copy(out_sh.at[pl.ds(s * orows + off, m)],
                            rows_a.at[pl.ds(0, m)])

            @pl.when(c == 0)
            def _(off=off, m=m):
                pltpu.sync_copy(rows_a.at[pl.ds(0, m)],
                                outp0_hbm.at[pl.ds(s * orows + off, m)])

            @pl.when(c == 1)
            def _(off=off, m=m):
                pltpu.sync_copy(rows_a.at[pl.ds(0, m)],
                                outp1_hbm.at[pl.ds(s * orows + off, m)])

            off += m

    return sc_gat


def kernel(x, edge_index, W, att_src, att_dst, bias):
    n, din = x.shape
    c = W.shape[1]
    e = edge_index.shape[1]
    ep = e + n
    nj = -(-ep // (32 * 128))
    nj = -(-nj // 8) * 8
    e_pad = 32 * nj * 128
    half = -(-(n // 2 + 1) // 128) * 128
    if 2 * half <= n:
        half += 128
    n_acc = 2 * half

    loops = jnp.arange(n, dtype=edge_index.dtype)
    ei = jnp.concatenate([edge_index, jnp.stack([loops, loops])], axis=1)
    pad = e_pad - ep
    src = jnp.concatenate([ei[0], jnp.zeros((pad,), jnp.int32)])
    pad_dst = n_acc + (jnp.arange(pad, dtype=jnp.int32) % TRASH)
    dst = jnp.concatenate([ei[1], pad_dst])
    src2 = src.reshape(e_pad // 128, 128)
    dst2 = dst.reshape(e_pad // 128, 128)

    xs, asrc, adst, g = _tc_prep(x, W, att_src.reshape(1, c),
                                 att_dst.reshape(1, c))
    n_logit = n_acc + TRASH
    asrc_p = jnp.concatenate(
        [jnp.pad(asrc[:, 0], (0, n_logit - n)),
         jnp.broadcast_to(g.reshape(1), (128,))]).reshape(
             n_logit // 128 + 1, 128)
    adst_p = jnp.pad(adst[:, 0], (0, n_logit - n)).reshape(n_logit // 128, 128)

    sc_gat = _make_sc_kernel(half, nj, c)
    att2, outp0, outp1 = sc_gat(src2, dst2, asrc_p, adst_p, xs)

    out = _tc_combine(outp0, outp1, bias.reshape(1, c))[:n]
    att = att2.reshape(e_pad)[:ep].reshape(ep, 1)
    return (out, (ei, att))
